# baseline (speedup 1.0000x reference)
import functools

import jax
import jax.numpy as jnp
from jax import lax
from jax.experimental import pallas as pl
from jax.experimental.pallas import tpu as pltpu
from jax.experimental.pallas import tpu_sc as plsc

N_NODES = 10000
N_PAD = 10240
N_EDGES = 320000
SUB = 32
NW = 32
CHUNK = 128
K_CHUNKS = 80
E_PER_W = K_CHUNKS * CHUNK
E_PAD = NW * E_PER_W
SLAB = N_PAD // 16
NBUF = 8
DIST = 6

_MESH = plsc.VectorSubcoreMesh(core_axis_name="c", subcore_axis_name="s")
_SC_PARAMS = pltpu.CompilerParams(use_tc_tiling_on_sc=False,
                                  needs_layout_passes=False)

_RSQRT_MAGIC = jnp.int32(0x5F3759DF)


def _rsqrt16(d):
    i = plsc.bitcast(d, jnp.int32)
    i = _RSQRT_MAGIC - lax.shift_right_logical(i, 1)
    y = plsc.bitcast(i, jnp.float32)
    half = 0.5 * d
    for _ in range(3):
        y = y * (1.5 - half * y * y)
    return y


@functools.partial(
    pl.kernel,
    out_type=jax.ShapeDtypeStruct((2, N_PAD), jnp.float32),
    mesh=_MESH,
    scratch_types=[
        pltpu.VMEM((K_CHUNKS, CHUNK), jnp.int32),
        pltpu.VMEM((CHUNK,), jnp.float32),
        pltpu.VMEM((SLAB,), jnp.float32),
        pltpu.VMEM_SHARED((N_PAD,), jnp.float32),
        pltpu.SemaphoreType.DMA,
    ],
    compiler_params=_SC_PARAMS,
)
def _deg_kernel(dst_hbm, out_hbm, idx_v, ones_v, zeros_v, deg_sh, sem):
    cid = lax.axis_index("c")
    sid = lax.axis_index("s")
    w = cid * 16 + sid

    pltpu.sync_copy(dst_hbm.at[w], idx_v)

    def fill(i, _):
        zeros_v[pl.ds(i * 16, 16)] = jnp.zeros((16,), jnp.float32)
        return 0
    lax.fori_loop(0, SLAB // 16, fill, 0)

    def fill1(i, _):
        ones_v[pl.ds(i * 16, 16)] = jnp.ones((16,), jnp.float32)
        return 0
    lax.fori_loop(0, CHUNK // 16, fill1, 0)

    pltpu.sync_copy(zeros_v, deg_sh.at[pl.ds(sid * SLAB, SLAB)])
    plsc.subcore_barrier()

    def body(g, _):
        for k in range(8):
            pltpu.async_copy(ones_v, deg_sh.at[idx_v.at[g * 8 + k]], sem,
                             add=True)
        for k in range(8):
            pltpu.make_async_copy(ones_v, deg_sh.at[idx_v.at[0]], sem).wait()
        return 0
    lax.fori_loop(0, K_CHUNKS // 8, body, 0)

    plsc.subcore_barrier()
    pltpu.sync_copy(deg_sh.at[pl.ds(sid * SLAB, SLAB)],
                    out_hbm.at[cid, pl.ds(sid * SLAB, SLAB)])


@functools.partial(
    pl.kernel,
    out_type=(
        jax.ShapeDtypeStruct((2, N_PAD, SUB), jnp.float32),
        jax.ShapeDtypeStruct((2, N_PAD), jnp.float32),
    ),
    mesh=_MESH,
    scratch_types=(
        [pltpu.VMEM((K_CHUNKS, CHUNK), jnp.int32)] * 2
        + [pltpu.VMEM((SLAB // 4, SUB), jnp.float32)]
        + [pltpu.VMEM((CHUNK, SUB), jnp.float32)] * NBUF
        + [pltpu.VMEM((64, SUB), jnp.float32)]
        + [pltpu.VMEM((SLAB,), jnp.float32)]
        + [pltpu.VMEM((SLAB,), jnp.float32)]
        + [pltpu.VMEM_SHARED((N_PAD, SUB), jnp.float32)]
        + [pltpu.VMEM_SHARED((N_PAD, SUB), jnp.float32)]
        + [pltpu.SemaphoreType.DMA] * (2 * NBUF)
    ),
    compiler_params=_SC_PARAMS,
)
def _scatter_kernel(src_hbm, dst_hbm, deg2_hbm, xs_hbm, z_hbm, dinv_hbm,
                    *scratch):
    src_v, dst_v, xs_v = scratch[:3]
    rows = scratch[3:3 + NBUF]
    zb_v, deg_v, dinv_v = scratch[3 + NBUF:6 + NBUF]
    z_sh, u_sh = scratch[6 + NBUF:8 + NBUF]
    gsem = scratch[8 + NBUF:8 + 2 * NBUF]
    ssem = scratch[8 + 2 * NBUF:8 + 3 * NBUF]

    cid = lax.axis_index("c")
    sid = lax.axis_index("s")
    w = cid * 16 + sid

    pltpu.sync_copy(src_hbm.at[w], src_v)
    pltpu.sync_copy(dst_hbm.at[w], dst_v)

    def fill(i, _):
        zb_v[i, pl.ds(0, 16)] = jnp.zeros((16,), jnp.float32)
        zb_v[i, pl.ds(16, 16)] = jnp.zeros((16,), jnp.float32)
        return 0
    lax.fori_loop(0, 64, fill, 0)

    def zslab(i, _):
        pltpu.sync_copy(zb_v, z_sh.at[pl.ds(sid * SLAB + i * 64, 64)])
        return 0
    lax.fori_loop(0, SLAB // 64, zslab, 0)

    pltpu.sync_copy(deg2_hbm.at[0, pl.ds(sid * SLAB, SLAB)], deg_v)
    pltpu.sync_copy(deg2_hbm.at[1, pl.ds(sid * SLAB, SLAB)], dinv_v)

    def mkdinv(g, _):
        s = pl.ds(g * 16, 16)
        dinv_v[s] = _rsqrt16(deg_v[s] + dinv_v[s] + 1.0)
        return 0
    lax.fori_loop(0, SLAB // 16, mkdinv, 0)

    qrows = SLAB // 4
    for q in range(4):
        base = sid * SLAB + q * qrows
        pltpu.sync_copy(xs_hbm.at[pl.ds(base, qrows)], xs_v)

        def scale(r, _):
            rg = q * qrows + r
            dv = plsc.load_gather(dinv_v, [jnp.full((16,), rg, jnp.int32)])
            xs_v[r, pl.ds(0, 16)] = xs_v[r, pl.ds(0, 16)] * dv
            xs_v[r, pl.ds(16, 16)] = xs_v[r, pl.ds(16, 16)] * dv
            return 0
        lax.fori_loop(0, qrows, scale, 0)
        pltpu.sync_copy(xs_v, u_sh.at[pl.ds(base, qrows)])

    pltpu.sync_copy(dinv_v, dinv_hbm.at[cid, pl.ds(sid * SLAB, SLAB)])
    plsc.subcore_barrier()

    for b in range(DIST):
        pltpu.async_copy(u_sh.at[src_v.at[b]], rows[b], gsem[b])

    def body(i, _):
        for b in range(NBUF):
            j = i * NBUF + b
            pltpu.make_async_copy(u_sh.at[src_v.at[j]], rows[b],
                                  gsem[b]).wait()
            pltpu.async_copy(rows[b], z_sh.at[dst_v.at[j]], ssem[b],
                             add=True)
            m = j + DIST
            bn = (b + DIST) % NBUF

            @pl.when(jnp.logical_and(m >= NBUF, m < K_CHUNKS))
            def _():
                pltpu.make_async_copy(rows[bn], z_sh.at[dst_v.at[0]],
                                      ssem[bn]).wait()

            @pl.when(m < K_CHUNKS)
            def _():
                pltpu.async_copy(u_sh.at[src_v.at[m]], rows[bn], gsem[bn])
        return 0
    lax.fori_loop(0, K_CHUNKS // NBUF, body, 0)

    for b in range(NBUF):
        pltpu.make_async_copy(rows[b], z_sh.at[dst_v.at[0]], ssem[b]).wait()

    plsc.subcore_barrier()

    pltpu.sync_copy(z_sh.at[pl.ds(sid * SLAB, SLAB)],
                    z_hbm.at[cid, pl.ds(sid * SLAB, SLAB)])


def _tc_body(dinv2_ref, z2_ref, xs_ref, w_ref, b_ref, out_ref):
    dinv = dinv2_ref[0][:, None]
    pre = dinv * (z2_ref[0] + z2_ref[1]) + dinv * dinv * xs_ref[...]
    h = jnp.dot(pre, w_ref[...], preferred_element_type=jnp.float32)
    out_ref[...] = jnp.maximum(h + b_ref[...], 0.0)


def _tc(dinv2, z2, xs_pad, wcat, bcat):
    return pl.pallas_call(
        _tc_body,
        out_shape=jax.ShapeDtypeStruct((N_PAD, 4 * SUB), jnp.float32),
    )(dinv2, z2, xs_pad, wcat, bcat)


@jax.jit
def kernel(x, edge_index, W, b):
    src = edge_index[0]
    dst = edge_index[1]
    pad = E_PAD - N_EDGES
    fill = jnp.full((pad,), N_NODES, jnp.int32)
    src_r = jnp.concatenate([src, fill]).reshape(NW, K_CHUNKS, CHUNK)
    dst_r = jnp.concatenate([dst, fill]).reshape(NW, K_CHUNKS, CHUNK)

    xs_pad = jnp.pad(x[:, :SUB], ((0, N_PAD - N_NODES), (0, 0)))

    deg2 = _deg_kernel(dst_r)
    z2, dinv2 = _scatter_kernel(src_r, dst_r, deg2, xs_pad)
    wcat = jnp.transpose(W, (1, 0, 2)).reshape(SUB, 4 * SUB)
    bcat = b.reshape(1, 4 * SUB)
    out = _tc(dinv2, z2, xs_pad, wcat, bcat)

    x_out = out[:N_NODES]
    heads = tuple(x_out[:, i * SUB:(i + 1) * SUB] for i in range(4))
    return (x_out,) + heads


# speedup vs baseline: 1.0063x; 1.0063x over previous
import functools

import jax
import jax.numpy as jnp
from jax import lax
from jax.experimental import pallas as pl
from jax.experimental.pallas import tpu as pltpu
from jax.experimental.pallas import tpu_sc as plsc

N_NODES = 10000
N_PAD = 10240
N_EDGES = 320000
SUB = 32
NW = 32
CHUNK = 128
K_CHUNKS = 80
E_PER_W = K_CHUNKS * CHUNK
E_PAD = NW * E_PER_W
SLAB = N_PAD // 16
NBUF = 8
DIST = 4

_MESH = plsc.VectorSubcoreMesh(core_axis_name="c", subcore_axis_name="s")
_SC_PARAMS = pltpu.CompilerParams(use_tc_tiling_on_sc=False)


@functools.partial(
    pl.kernel,
    out_type=jax.ShapeDtypeStruct((2, N_PAD), jnp.float32),
    mesh=_MESH,
    scratch_types=[
        pltpu.VMEM((K_CHUNKS, CHUNK), jnp.int32),
        pltpu.VMEM((CHUNK,), jnp.float32),
        pltpu.VMEM((SLAB,), jnp.float32),
        pltpu.VMEM_SHARED((N_PAD,), jnp.float32),
        pltpu.SemaphoreType.DMA,
    ],
    compiler_params=_SC_PARAMS,
)
def _deg_kernel(dst_hbm, out_hbm, idx_v, ones_v, zeros_v, deg_sh, sem):
    cid = lax.axis_index("c")
    sid = lax.axis_index("s")
    w = cid * 16 + sid

    pltpu.sync_copy(dst_hbm.at[w], idx_v)

    def fill(i, _):
        zeros_v[pl.ds(i * 16, 16)] = jnp.zeros((16,), jnp.float32)
        return 0
    lax.fori_loop(0, SLAB // 16, fill, 0)

    def fill1(i, _):
        ones_v[pl.ds(i * 16, 16)] = jnp.ones((16,), jnp.float32)
        return 0
    lax.fori_loop(0, CHUNK // 16, fill1, 0)

    pltpu.sync_copy(zeros_v, deg_sh.at[pl.ds(sid * SLAB, SLAB)])
    plsc.subcore_barrier()

    def body(g, _):
        for k in range(8):
            pltpu.async_copy(ones_v, deg_sh.at[idx_v.at[g * 8 + k]], sem,
                             add=True)
        for k in range(8):
            pltpu.make_async_copy(ones_v, deg_sh.at[idx_v.at[0]], sem).wait()
        return 0
    lax.fori_loop(0, K_CHUNKS // 8, body, 0)

    plsc.subcore_barrier()
    pltpu.sync_copy(deg_sh.at[pl.ds(sid * SLAB, SLAB)],
                    out_hbm.at[cid, pl.ds(sid * SLAB, SLAB)])


@functools.partial(
    pl.kernel,
    out_type=jax.ShapeDtypeStruct((2, N_PAD, SUB), jnp.float32),
    mesh=_MESH,
    scratch_types=(
        [pltpu.VMEM((K_CHUNKS, CHUNK), jnp.int32)] * 2
        + [pltpu.VMEM((CHUNK, SUB), jnp.float32)] * NBUF
        + [pltpu.VMEM((64, SUB), jnp.float32)]
        + [pltpu.VMEM_SHARED((N_PAD, SUB), jnp.float32)]
        + [pltpu.VMEM_SHARED((N_PAD, SUB), jnp.float32)]
        + [pltpu.SemaphoreType.DMA] * (2 * NBUF)
    ),
    compiler_params=_SC_PARAMS,
)
def _scatter_kernel(src_hbm, dst_hbm, u_hbm, out_hbm, *scratch):
    src_v, dst_v = scratch[0], scratch[1]
    rows = scratch[2:2 + NBUF]
    zb_v = scratch[2 + NBUF]
    z_sh = scratch[3 + NBUF]
    u_sh = scratch[4 + NBUF]
    gsem = scratch[5 + NBUF:5 + 2 * NBUF]
    ssem = scratch[5 + 2 * NBUF:5 + 3 * NBUF]

    cid = lax.axis_index("c")
    sid = lax.axis_index("s")
    w = cid * 16 + sid

    pltpu.sync_copy(src_hbm.at[w], src_v)
    pltpu.sync_copy(dst_hbm.at[w], dst_v)
    pltpu.sync_copy(u_hbm.at[pl.ds(sid * SLAB, SLAB)],
                    u_sh.at[pl.ds(sid * SLAB, SLAB)])

    def fill(i, _):
        zb_v[i, pl.ds(0, 16)] = jnp.zeros((16,), jnp.float32)
        zb_v[i, pl.ds(16, 16)] = jnp.zeros((16,), jnp.float32)
        return 0
    lax.fori_loop(0, 64, fill, 0)

    def zslab(i, _):
        pltpu.sync_copy(zb_v, z_sh.at[pl.ds(sid * SLAB + i * 64, 64)])
        return 0
    lax.fori_loop(0, SLAB // 64, zslab, 0)
    plsc.subcore_barrier()

    for b in range(DIST):
        pltpu.async_copy(u_sh.at[src_v.at[b]], rows[b], gsem[b])

    def body(i, _):
        for b in range(NBUF):
            j = i * NBUF + b
            pltpu.make_async_copy(u_sh.at[src_v.at[j]], rows[b],
                                  gsem[b]).wait()
            pltpu.async_copy(rows[b], z_sh.at[dst_v.at[j]], ssem[b], add=True)
            m = j + DIST
            bn = (b + DIST) % NBUF

            @pl.when(jnp.logical_and(m >= NBUF, m < K_CHUNKS))
            def _():
                pltpu.make_async_copy(rows[bn], z_sh.at[dst_v.at[0]],
                                      ssem[bn]).wait()

            @pl.when(m < K_CHUNKS)
            def _():
                pltpu.async_copy(u_sh.at[src_v.at[m]], rows[bn], gsem[bn])
        return 0
    lax.fori_loop(0, K_CHUNKS // NBUF, body, 0)

    for b in range(NBUF):
        pltpu.make_async_copy(rows[b], z_sh.at[dst_v.at[0]], ssem[b]).wait()

    plsc.subcore_barrier()
    pltpu.sync_copy(z_sh.at[pl.ds(sid * SLAB, SLAB)],
                    out_hbm.at[cid, pl.ds(sid * SLAB, SLAB)])


def _tc1_body(deg2_ref, xs_ref, u_ref):
    deg = deg2_ref[0] + deg2_ref[1] + 1.0
    dinv = lax.rsqrt(deg)
    u_ref[...] = xs_ref[...] * dinv[:, None]


def _tc1(deg2, xs_pad):
    return pl.pallas_call(
        _tc1_body,
        out_shape=jax.ShapeDtypeStruct((N_PAD, SUB), jnp.float32),
    )(deg2, xs_pad)


def _tc2_body(deg2_ref, z2_ref, u_ref, w_ref, b_ref, out_ref):
    deg = deg2_ref[0] + deg2_ref[1] + 1.0
    dinv = lax.rsqrt(deg)[:, None]
    pre = dinv * (z2_ref[0] + z2_ref[1] + u_ref[...])
    h = jnp.dot(pre, w_ref[...], preferred_element_type=jnp.float32)
    out_ref[...] = jnp.maximum(h + b_ref[...], 0.0)


def _tc2(deg2, z2, u, wcat, bcat):
    return pl.pallas_call(
        _tc2_body,
        out_shape=jax.ShapeDtypeStruct((N_PAD, 4 * SUB), jnp.float32),
    )(deg2, z2, u, wcat, bcat)


@jax.jit
def kernel(x, edge_index, W, b):
    src = edge_index[0]
    dst = edge_index[1]
    pad = E_PAD - N_EDGES
    fill = jnp.full((pad,), N_NODES, jnp.int32)
    src_r = jnp.concatenate([src, fill]).reshape(NW, K_CHUNKS, CHUNK)
    dst_r = jnp.concatenate([dst, fill]).reshape(NW, K_CHUNKS, CHUNK)

    xs_pad = jnp.pad(x[:, :SUB], ((0, N_PAD - N_NODES), (0, 0)))

    deg2 = _deg_kernel(dst_r)
    u = _tc1(deg2, xs_pad)
    z2 = _scatter_kernel(src_r, dst_r, u)
    wcat = jnp.transpose(W, (1, 0, 2)).reshape(SUB, 4 * SUB)
    bcat = b.reshape(1, 4 * SUB)
    out = _tc2(deg2, z2, u, wcat, bcat)

    x_out = out[:N_NODES]
    heads = tuple(x_out[:, i * SUB:(i + 1) * SUB] for i in range(4))
    return (x_out,) + heads
